# Initial kernel scaffold; baseline (speedup 1.0000x reference)
import jax
import jax.numpy as jnp
from jax.experimental import pallas as pl


def kernel(logits, targets, logit_lens, target_lens):
    raise NotImplementedError("write your pallas kernel here")


# speedup vs baseline: 400.6918x; 400.6918x over previous
import jax
import jax.numpy as jnp
from jax import lax
from jax.experimental import pallas as pl
from jax.experimental.pallas import tpu as pltpu

_T = 64
_U = 16
_D = 1024
_NEG = -1e30


def _shr(x, k, fill):
    pad = jnp.full(x.shape[:-1] + (k,), fill, x.dtype)
    return jnp.concatenate([pad, x[..., : x.shape[-1] - k]], axis=-1)


def _body(x_ref, tgt_ref, tl_ref, o_la, o_st, o_tot):
    x = x_ref[:]
    tl = tl_ref[0]

    m = jnp.max(x, axis=2)
    e = jnp.exp(x - m[:, :, None])
    logs = jnp.log(jnp.sum(e, axis=2))
    colio = lax.broadcasted_iota(jnp.int32, (_U, _D), 1)
    mask = colio == tgt_ref[:]
    g_raw = jnp.sum(jnp.where(mask[None], x, 0.0), axis=2)
    gath = (g_raw - m) - logs
    bl = (x[:, :, 0] - m) - logs

    gT = gath.T
    bT = bl.T

    z = bT
    for k in (1, 2, 4, 8, 16, 32):
        z = z + _shr(z, k, 0.0)
    bc = _shr(z, 1, 0.0)

    tar = lax.broadcasted_iota(jnp.int32, (1, _T), 1).astype(jnp.float32)

    la = jnp.zeros((1, _T), jnp.float32)
    st = tar
    tot = jnp.ones((1, _T), jnp.float32)
    acc_la, acc_st, acc_tot = la, st, tot
    acc_bl = bT[0:1]

    for u in range(1, _U):
        g_row = gT[u - 1 : u]
        bc_row = bc[u : u + 1]
        v = la + g_row - bc_row
        s_st = st
        s_tot = tot - tar
        for k in (1, 2, 4, 8, 16, 32):
            vs = _shr(v, k, _NEG)
            ss = _shr(s_st, k, 0.0)
            ts = _shr(s_tot, k, 0.0)
            keep = vs >= v
            v = jnp.where(keep, vs, v)
            s_st = jnp.where(keep, ss, s_st)
            s_tot = jnp.where(keep, ts, s_tot)
        la = v + bc_row
        st = s_st
        tot = s_tot + tar + 1.0
        pred = tl == u
        acc_la = jnp.where(pred, la, acc_la)
        acc_st = jnp.where(pred, st, acc_st)
        acc_tot = jnp.where(pred, tot, acc_tot)
        acc_bl = jnp.where(pred, bT[u : u + 1], acc_bl)

    o_la[:] = acc_la + acc_bl
    o_st[:] = acc_st
    o_tot[:] = acc_tot + 1.0


def kernel(logits, targets, logit_lens, target_lens):
    x = logits[0]
    tgt = targets.reshape(_U, 1).astype(jnp.int32)
    tl = target_lens.astype(jnp.int32)

    out_shape = [jax.ShapeDtypeStruct((1, _T), jnp.float32)] * 3
    la_each, st_each, tot_each = pl.pallas_call(
        _body,
        out_shape=out_shape,
        in_specs=[
            pl.BlockSpec((_T, _U, _D), lambda: (0, 0, 0)),
            pl.BlockSpec((_U, 1), lambda: (0, 0)),
            pl.BlockSpec(memory_space=pltpu.SMEM),
        ],
        out_specs=[pl.BlockSpec((1, _T), lambda: (0, 0))] * 3,
    )(x, tgt, tl)

    la_each = la_each.reshape(_T)
    st_each = st_each.reshape(_T)
    tot_each = tot_each.reshape(_T)
    return (la_each[_T - 1], la_each, st_each, tot_each)


# speedup vs baseline: 485.4824x; 1.2116x over previous
import jax
import jax.numpy as jnp
from jax import lax
from jax.experimental import pallas as pl
from jax.experimental.pallas import tpu as pltpu

_T = 64
_U = 16
_D = 1024
_L = 128
_NEG = -1e30


def _body(x_ref, tgt_ref, tl_ref, o_lp, o_la, o_st, o_tot):
    x = x_ref[0]
    tl = tl_ref[0]

    m = jnp.max(x, axis=2)
    e = jnp.exp(x - m[:, :, None])
    logs = jnp.log(jnp.sum(e, axis=2))
    colio = lax.broadcasted_iota(jnp.int32, (_U, _D), 1)
    mask = colio == tgt_ref[:].reshape(1, _U).T
    g_raw = jnp.sum(jnp.where(mask[None], x, 0.0), axis=2)
    gath = (g_raw - m) - logs
    bl = (x[:, :, 0] - m) - logs

    zpad = jnp.zeros((_U, _L - _T), jnp.float32)
    gT = jnp.concatenate([gath.T, zpad], axis=1)
    bT = jnp.concatenate([bl.T, zpad], axis=1)

    lio = lax.broadcasted_iota(jnp.int32, (1, _L), 1)
    lio2 = lax.broadcasted_iota(jnp.int32, (2, _L), 1)
    z = jnp.where(lio < _T, bT, 0.0)
    for k in (1, 2, 4, 8, 16, 32):
        z = z + jnp.where(lio < k, 0.0, pltpu.roll(z, k, 1))
    bc = jnp.where(lio < 1, 0.0, pltpu.roll(z, 1, 1))

    tar = lio.astype(jnp.float32)

    la = jnp.where(lio < _T, 0.0, _NEG)
    st = tar
    tot = jnp.ones((1, _L), jnp.float32)
    acc_la, acc_st, acc_tot = la, st, tot
    acc_bl = bT[0:1]

    for u in range(1, _U):
        g_row = gT[u - 1 : u]
        bc_row = bc[u : u + 1]
        v = la + g_row - bc_row
        V = jnp.concatenate([v, v], axis=0)
        P = jnp.concatenate([st, tot - tar], axis=0)
        for k in (1, 2, 4, 8, 16, 32):
            Vs = jnp.where(lio2 < k, _NEG, pltpu.roll(V, k, 1))
            Ps = pltpu.roll(P, k, 1)
            keep = Vs >= V
            V = jnp.where(keep, Vs, V)
            P = jnp.where(keep, Ps, P)
        la = V[0:1] + bc_row
        st = P[0:1]
        tot = P[1:2] + tar + 1.0
        pred = tl == u
        acc_la = jnp.where(pred, la, acc_la)
        acc_st = jnp.where(pred, st, acc_st)
        acc_tot = jnp.where(pred, tot, acc_tot)
        acc_bl = jnp.where(pred, bT[u : u + 1], acc_bl)

    la_each = acc_la + acc_bl
    o_lp[:] = la_each[:, _T - 1 : _T]
    o_la[:] = la_each[:, :_T]
    o_st[:] = acc_st[:, :_T]
    o_tot[:] = acc_tot[:, :_T] + 1.0


def kernel(logits, targets, logit_lens, target_lens):
    tl = target_lens.astype(jnp.int32)

    out_shape = [
        jax.ShapeDtypeStruct((1, 1), jnp.float32),
        jax.ShapeDtypeStruct((1, _T), jnp.float32),
        jax.ShapeDtypeStruct((1, _T), jnp.float32),
        jax.ShapeDtypeStruct((1, _T), jnp.float32),
    ]
    lp, la_each, st_each, tot_each = pl.pallas_call(
        _body,
        out_shape=out_shape,
        in_specs=[
            pl.BlockSpec((1, _T, _U, _D), lambda: (0, 0, 0, 0)),
            pl.BlockSpec((1, _U), lambda: (0, 0)),
            pl.BlockSpec(memory_space=pltpu.SMEM),
        ],
        out_specs=[
            pl.BlockSpec((1, 1), lambda: (0, 0)),
            pl.BlockSpec((1, _T), lambda: (0, 0)),
            pl.BlockSpec((1, _T), lambda: (0, 0)),
            pl.BlockSpec((1, _T), lambda: (0, 0)),
        ],
    )(logits, targets.astype(jnp.int32), tl)

    return (
        lp.reshape(()),
        la_each.reshape(_T),
        st_each.reshape(_T),
        tot_each.reshape(_T),
    )
